# baseline (speedup 1.0000x reference)
import jax
import jax.numpy as jnp
from jax import lax
from jax.experimental import pallas as pl
from jax.experimental.pallas import tpu as pltpu
from jax.experimental.pallas import tpu_sc as plsc

_B = 16384
_D = 64
_V = 1_000_000
_NC = 2
_NS = 16
_NW = _NC * _NS
_BPW = _B // _NW
_CH = 128
_NCH = _BPW // _CH
_NG = _CH // 16

_TCOLS = _V // 128
_TAIL = _V - _TCOLS * 128
_ITER = (_TCOLS + _NW - 1) // _NW


def _rsqrt(x):
    i = lax.bitcast_convert_type(x, jnp.int32)
    seed = jnp.int32(0x5F3759DF) - lax.shift_right_logical(i, 1)
    y = lax.bitcast_convert_type(seed, jnp.float32)
    for _ in range(3):
        y = y * (1.5 - 0.5 * x * y * y)
    return y


def _diag_bases(iot):
    zero = jnp.zeros((16,), jnp.int32)
    ib, ob = [], []
    for s in range(16):
        t = jnp.bitwise_and(iot + s, 15)
        ib.append(iot * 128 + t)
        ob.append(lax.shift_right_logical(t, 1) * 128
                  + lax.shift_left(jnp.bitwise_and(t, 1), 6) + iot)
    return zero, ib, ob


def _transpose_block(inb, outb, bases):
    zero, ib, ob = bases

    def sub(sb, carry):
        d0 = jnp.bitwise_and(sb, 3) * 16
        e0 = lax.shift_right_logical(sb, 2) * 16
        ioff = d0 * 128 + e0
        ooff = lax.shift_right_logical(e0, 1) * 128 + d0
        gs = [plsc.load_gather(inb, [zero, ib[s] + ioff]) for s in range(16)]
        for s in range(16):
            plsc.store_scatter(outb, [zero, ob[s] + ooff], gs[s])
        return carry

    lax.fori_loop(0, 32, sub, 0)


def _trans_body(ent_t, tail32, out, inbuf, outbuf, sin0, sin1, sout0, sout1):
    wid = lax.axis_index("s") * _NC + lax.axis_index("c")
    iot = lax.iota(jnp.int32, 16)
    sin = (sin0, sin1)
    sout = (sout0, sout1)
    bases = _diag_bases(iot)

    def in_copy(k, b):
        cc = wid + k * _NW

        @pl.when(cc < _TCOLS)
        def _():
            pltpu.async_copy(ent_t.at[:, pl.ds(cc * 128, 128)],
                             inbuf.at[b], sin[b])

    def in_wait(k, b):
        cc = wid + k * _NW

        @pl.when(cc < _TCOLS)
        def _():
            pltpu.make_async_copy(ent_t.at[:, pl.ds(cc * 128, 128)],
                                  inbuf.at[b], sin[b]).wait()

    def out_copy(k, b):
        cc = wid + k * _NW

        @pl.when(cc < _TCOLS)
        def _():
            pltpu.async_copy(outbuf.at[b], out.at[pl.ds(cc * 64, 64)],
                             sout[b])

    def out_wait(k, b):
        cc = wid + k * _NW

        @pl.when(cc < _TCOLS)
        def _():
            pltpu.make_async_copy(outbuf.at[b], out.at[pl.ds(cc * 64, 64)],
                                  sout[b]).wait()

    in_copy(0, 0)

    def step(i, carry):
        for b in range(2):
            k = 2 * i + b
            in_copy(k + 1, 1 - b)
            in_wait(k, b)

            @pl.when(k >= 2)
            def _():
                out_wait(k - 2, b)

            @pl.when(wid + k * _NW < _TCOLS)
            def _():
                _transpose_block(inbuf.at[b], outbuf.at[b], bases)

            out_copy(k, b)
        return carry

    lax.fori_loop(0, (_ITER + 1) // 2, step, 0)
    for k in (_ITER - 1, _ITER):
        out_wait(k, k % 2)

    @pl.when(wid == 0)
    def _():
        pltpu.sync_copy(tail32, outbuf.at[0].at[pl.ds(0, _TAIL // 2)])
        pltpu.sync_copy(outbuf.at[0].at[pl.ds(0, _TAIL // 2)],
                        out.at[pl.ds(_TCOLS * 64, _TAIL // 2)])


def _score_body(ph, pr, pt, nh, nr, nt, ent2, rel2, p_out, n_out,
                ia, ib, ic, ja, jb, jc, abuf, bbuf, cbuf, obuf, sem):
    wid = lax.axis_index("s") * _NC + lax.axis_index("c")
    iot = lax.iota(jnp.int32, 16)

    for idx_a, tab_a, idx_b, tab_b, idx_c, tab_c, out in (
        (ph, ent2, pr, rel2, pt, ent2, p_out),
        (nh, ent2, nt, ent2, nr, rel2, n_out),
    ):
        row0 = wid * _NCH

        def chunk_body(c, carry):
            pltpu.sync_copy(idx_a.at[pl.ds(row0 + c, 1)], ia)
            pltpu.sync_copy(idx_b.at[pl.ds(row0 + c, 1)], ib)
            pltpu.sync_copy(idx_c.at[pl.ds(row0 + c, 1)], ic)
            for src, dst in ((ia, ja), (ib, jb), (ic, jc)):
                for k in range(_CH // 16):
                    sl = pl.ds(k * 16, 16)
                    dst[0, sl] = lax.shift_right_logical(src[0, sl], 1)
            da = pltpu.async_copy(tab_a.at[ja.at[0]], abuf, sem)
            db = pltpu.async_copy(tab_b.at[jb.at[0]], bbuf, sem)
            dc = pltpu.async_copy(tab_c.at[jc.at[0]], cbuf, sem)
            da.wait()
            db.wait()
            dc.wait()

            def group(g, inner):
                r = g * 16 + iot
                zi = jnp.zeros((16,), jnp.int32)
                ha = jnp.bitwise_and(plsc.load_gather(ia, [zi, r]), 1) * _D
                hb = jnp.bitwise_and(plsc.load_gather(ib, [zi, r]), 1) * _D
                hc = jnp.bitwise_and(plsc.load_gather(ic, [zi, r]), 1) * _D
                r128 = r * 128
                fa = r128 + ha
                fb = r128 + hb
                fc = r128 + hc
                z = jnp.zeros((16,), jnp.float32)
                aa, bb, cc, ab, ac, bc = z, z, z, z, z, z
                for d in range(_D):
                    col = jnp.bitwise_and(iot + d, _D - 1)
                    av = plsc.load_gather(abuf, [zi, fa + col])
                    bv = plsc.load_gather(bbuf, [zi, fb + col])
                    cv = plsc.load_gather(cbuf, [zi, fc + col])
                    aa += av * av
                    bb += bv * bv
                    cc += cv * cv
                    ab += av * bv
                    ac += av * cv
                    bc += bv * cv
                inva = _rsqrt(jnp.maximum(aa, 1e-24))
                invb = _rsqrt(jnp.maximum(bb, 1e-24))
                invc = _rsqrt(jnp.maximum(cc, 1e-24))
                s2 = 3.0 + 2.0 * (ab * inva * invb - ac * inva * invc
                                  - bc * invb * invc)
                s2 = jnp.maximum(s2, 0.0)
                score = s2 * _rsqrt(jnp.maximum(s2, 1e-30))
                obuf[pl.ds(c * _CH + g * 16, 16)] = score
                return inner

            lax.fori_loop(0, _NG, group, 0)
            return carry

        lax.fori_loop(0, _NCH, chunk_body, 0)
        pltpu.sync_copy(obuf, out.at[pl.ds(wid * _BPW, _BPW)])


def kernel(pos_h, pos_r, pos_t, neg_h, neg_r, neg_t, ent_emb, rel_emb):
    shp = (_B // _CH, _CH)
    ph = pos_h.astype(jnp.int32).reshape(shp)
    pr = pos_r.astype(jnp.int32).reshape(shp)
    pt = pos_t.astype(jnp.int32).reshape(shp)
    nh = neg_h.astype(jnp.int32).reshape(shp)
    nr = neg_r.astype(jnp.int32).reshape(shp)
    nt = neg_t.astype(jnp.int32).reshape(shp)
    ent_t = jnp.transpose(ent_emb)
    tail32 = ent_emb[_TCOLS * 128:].reshape(_TAIL // 2, 2 * _D)
    rel2 = rel_emb.reshape(-1, 2 * _D)

    mesh = plsc.VectorSubcoreMesh(core_axis_name="c", subcore_axis_name="s")
    cparams = pltpu.CompilerParams(
        use_tc_tiling_on_sc=True,
        needs_layout_passes=False,
        disable_bounds_checks=True,
    )

    transpose_run = pl.kernel(
        _trans_body,
        mesh=mesh,
        compiler_params=cparams,
        out_type=[jax.ShapeDtypeStruct((_V // 2, 2 * _D), jnp.float32)],
        scratch_types=[
            pltpu.VMEM((2, _D, 128), jnp.float32),
            pltpu.VMEM((2, _D, 128), jnp.float32),
            pltpu.SemaphoreType.DMA,
            pltpu.SemaphoreType.DMA,
            pltpu.SemaphoreType.DMA,
            pltpu.SemaphoreType.DMA,
        ],
    )
    (ent2,) = transpose_run(ent_t, tail32)

    score_run = pl.kernel(
        _score_body,
        mesh=mesh,
        compiler_params=cparams,
        out_type=[
            jax.ShapeDtypeStruct((_B,), jnp.float32),
            jax.ShapeDtypeStruct((_B,), jnp.float32),
        ],
        scratch_types=[
            pltpu.VMEM((1, _CH), jnp.int32),
            pltpu.VMEM((1, _CH), jnp.int32),
            pltpu.VMEM((1, _CH), jnp.int32),
            pltpu.VMEM((1, _CH), jnp.int32),
            pltpu.VMEM((1, _CH), jnp.int32),
            pltpu.VMEM((1, _CH), jnp.int32),
            pltpu.VMEM((_CH, 2 * _D), jnp.float32),
            pltpu.VMEM((_CH, 2 * _D), jnp.float32),
            pltpu.VMEM((_CH, 2 * _D), jnp.float32),
            pltpu.VMEM((_BPW,), jnp.float32),
            pltpu.SemaphoreType.DMA,
        ],
    )
    p_score, n_score = score_run(ph, pr, pt, nh, nr, nt, ent2, rel2)
    return (p_score, n_score)


# speedup vs baseline: 1.1941x; 1.1941x over previous
import jax
import jax.numpy as jnp
from jax import lax
from jax.experimental import pallas as pl
from jax.experimental.pallas import tpu as pltpu
from jax.experimental.pallas import tpu_sc as plsc

_B = 16384
_D = 64
_V = 1_000_000
_NC = 2
_NS = 16
_NW = _NC * _NS
_BPW = _B // _NW
_CH = 128
_NCH = _BPW // _CH
_NG = _CH // 16

_TCOLS = _V // 128
_TAIL = _V - _TCOLS * 128
_ITER = (_TCOLS + _NW - 1) // _NW


def _rsqrt(x):
    i = lax.bitcast_convert_type(x, jnp.int32)
    seed = jnp.int32(0x5F3759DF) - lax.shift_right_logical(i, 1)
    y = lax.bitcast_convert_type(seed, jnp.float32)
    for _ in range(3):
        y = y * (1.5 - 0.5 * x * y * y)
    return y


def _diag_bases(iot):
    zero = jnp.zeros((16,), jnp.int32)
    ib, ob = [], []
    for s in range(16):
        t = jnp.bitwise_and(iot + s, 15)
        ib.append(iot * 128 + t)
        ob.append(lax.shift_right_logical(t, 1) * 128
                  + lax.shift_left(jnp.bitwise_and(t, 1), 6) + iot)
    return zero, ib, ob


def _transpose_block(inb, outb, bases):
    zero, ib, ob = bases

    def sub(sb, carry):
        d0 = jnp.bitwise_and(sb, 3) * 16
        e0 = lax.shift_right_logical(sb, 2) * 16
        ioff = d0 * 128 + e0
        ooff = lax.shift_right_logical(e0, 1) * 128 + d0
        gs = [plsc.load_gather(inb, [zero, ib[s] + ioff]) for s in range(16)]
        for s in range(16):
            plsc.store_scatter(outb, [zero, ob[s] + ooff], gs[s])
        return carry

    lax.fori_loop(0, 32, sub, 0)


def _trans_body(ent_t, tail32, out, inbuf, outbuf,
                sin0, sin1, sin2, sin3, sout0, sout1):
    wid = lax.axis_index("s") * _NC + lax.axis_index("c")
    iot = lax.iota(jnp.int32, 16)
    sin = (sin0, sin1, sin2, sin3)
    sout = (sout0, sout1)
    bases = _diag_bases(iot)

    def in_copy(k, b):
        cc = wid + k * _NW

        @pl.when(cc < _TCOLS)
        def _():
            pltpu.async_copy(ent_t.at[:, pl.ds(cc * 128, 128)],
                             inbuf.at[b], sin[b])

    def in_wait(k, b):
        cc = wid + k * _NW

        @pl.when(cc < _TCOLS)
        def _():
            pltpu.make_async_copy(ent_t.at[:, pl.ds(cc * 128, 128)],
                                  inbuf.at[b], sin[b]).wait()

    def out_copy(k, b):
        cc = wid + k * _NW

        @pl.when(cc < _TCOLS)
        def _():
            pltpu.async_copy(outbuf.at[b], out.at[pl.ds(cc * 64, 64)],
                             sout[b])

    def out_wait(k, b):
        cc = wid + k * _NW

        @pl.when(cc < _TCOLS)
        def _():
            pltpu.make_async_copy(outbuf.at[b], out.at[pl.ds(cc * 64, 64)],
                                  sout[b]).wait()

    in_copy(0, 0)
    in_copy(1, 1)

    def step(i, carry):
        for b in range(4):
            k = 4 * i + b
            in_copy(k + 2, (b + 2) % 4)
            in_wait(k, b)

            @pl.when(k >= 2)
            def _():
                out_wait(k - 2, b % 2)

            @pl.when(wid + k * _NW < _TCOLS)
            def _():
                _transpose_block(inbuf.at[b], outbuf.at[b % 2], bases)

            out_copy(k, b % 2)
        return carry

    lax.fori_loop(0, (_ITER + 3) // 4, step, 0)

    @pl.when(wid == 0)
    def _():
        pltpu.sync_copy(tail32, outbuf.at[0].at[pl.ds(0, _TAIL // 2)])
        pltpu.sync_copy(outbuf.at[0].at[pl.ds(0, _TAIL // 2)],
                        out.at[pl.ds(_TCOLS * 64, _TAIL // 2)])


def _score_body(ph, pr, pt, nh, nr, nt, ent2, rel2, p_out, n_out,
                ia, ib, ic, ja, jb, jc, abuf, bbuf, cbuf, obuf, sem):
    wid = lax.axis_index("s") * _NC + lax.axis_index("c")
    iot = lax.iota(jnp.int32, 16)

    for idx_a, tab_a, idx_b, tab_b, idx_c, tab_c, out in (
        (ph, ent2, pr, rel2, pt, ent2, p_out),
        (nh, ent2, nt, ent2, nr, rel2, n_out),
    ):
        row0 = wid * _NCH

        def chunk_body(c, carry):
            pltpu.sync_copy(idx_a.at[pl.ds(row0 + c, 1)], ia)
            pltpu.sync_copy(idx_b.at[pl.ds(row0 + c, 1)], ib)
            pltpu.sync_copy(idx_c.at[pl.ds(row0 + c, 1)], ic)
            for src, dst in ((ia, ja), (ib, jb), (ic, jc)):
                for k in range(_CH // 16):
                    sl = pl.ds(k * 16, 16)
                    dst[0, sl] = lax.shift_right_logical(src[0, sl], 1)
            da = pltpu.async_copy(tab_a.at[ja.at[0]], abuf, sem)
            db = pltpu.async_copy(tab_b.at[jb.at[0]], bbuf, sem)
            dc = pltpu.async_copy(tab_c.at[jc.at[0]], cbuf, sem)
            da.wait()
            db.wait()
            dc.wait()

            def group(g, inner):
                r = g * 16 + iot
                zi = jnp.zeros((16,), jnp.int32)
                ha = jnp.bitwise_and(plsc.load_gather(ia, [zi, r]), 1) * _D
                hb = jnp.bitwise_and(plsc.load_gather(ib, [zi, r]), 1) * _D
                hc = jnp.bitwise_and(plsc.load_gather(ic, [zi, r]), 1) * _D
                r128 = r * 128
                fa = r128 + ha
                fb = r128 + hb
                fc = r128 + hc
                z = jnp.zeros((16,), jnp.float32)
                aa, bb, cc, ab, ac, bc = z, z, z, z, z, z
                for d in range(_D):
                    col = jnp.bitwise_and(iot + d, _D - 1)
                    av = plsc.load_gather(abuf, [zi, fa + col])
                    bv = plsc.load_gather(bbuf, [zi, fb + col])
                    cv = plsc.load_gather(cbuf, [zi, fc + col])
                    aa += av * av
                    bb += bv * bv
                    cc += cv * cv
                    ab += av * bv
                    ac += av * cv
                    bc += bv * cv
                inva = _rsqrt(jnp.maximum(aa, 1e-24))
                invb = _rsqrt(jnp.maximum(bb, 1e-24))
                invc = _rsqrt(jnp.maximum(cc, 1e-24))
                s2 = 3.0 + 2.0 * (ab * inva * invb - ac * inva * invc
                                  - bc * invb * invc)
                s2 = jnp.maximum(s2, 0.0)
                score = s2 * _rsqrt(jnp.maximum(s2, 1e-30))
                obuf[pl.ds(c * _CH + g * 16, 16)] = score
                return inner

            lax.fori_loop(0, _NG, group, 0)
            return carry

        lax.fori_loop(0, _NCH, chunk_body, 0)
        pltpu.sync_copy(obuf, out.at[pl.ds(wid * _BPW, _BPW)])


def kernel(pos_h, pos_r, pos_t, neg_h, neg_r, neg_t, ent_emb, rel_emb):
    shp = (_B // _CH, _CH)
    ph = pos_h.astype(jnp.int32).reshape(shp)
    pr = pos_r.astype(jnp.int32).reshape(shp)
    pt = pos_t.astype(jnp.int32).reshape(shp)
    nh = neg_h.astype(jnp.int32).reshape(shp)
    nr = neg_r.astype(jnp.int32).reshape(shp)
    nt = neg_t.astype(jnp.int32).reshape(shp)
    ent_t = jnp.transpose(ent_emb)
    tail32 = ent_emb[_TCOLS * 128:].reshape(_TAIL // 2, 2 * _D)
    rel2 = rel_emb.reshape(-1, 2 * _D)

    mesh = plsc.VectorSubcoreMesh(core_axis_name="c", subcore_axis_name="s")
    cparams = pltpu.CompilerParams(
        use_tc_tiling_on_sc=True,
        needs_layout_passes=False,
        disable_bounds_checks=True,
    )

    transpose_run = pl.kernel(
        _trans_body,
        mesh=mesh,
        compiler_params=cparams,
        out_type=[jax.ShapeDtypeStruct((_V // 2, 2 * _D), jnp.float32)],
        scratch_types=[
            pltpu.VMEM((4, _D, 128), jnp.float32),
            pltpu.VMEM((2, _D, 128), jnp.float32),
            pltpu.SemaphoreType.DMA,
            pltpu.SemaphoreType.DMA,
            pltpu.SemaphoreType.DMA,
            pltpu.SemaphoreType.DMA,
            pltpu.SemaphoreType.DMA,
            pltpu.SemaphoreType.DMA,
        ],
    )
    (ent2,) = transpose_run(ent_t, tail32)

    score_run = pl.kernel(
        _score_body,
        mesh=mesh,
        compiler_params=cparams,
        out_type=[
            jax.ShapeDtypeStruct((_B,), jnp.float32),
            jax.ShapeDtypeStruct((_B,), jnp.float32),
        ],
        scratch_types=[
            pltpu.VMEM((1, _CH), jnp.int32),
            pltpu.VMEM((1, _CH), jnp.int32),
            pltpu.VMEM((1, _CH), jnp.int32),
            pltpu.VMEM((1, _CH), jnp.int32),
            pltpu.VMEM((1, _CH), jnp.int32),
            pltpu.VMEM((1, _CH), jnp.int32),
            pltpu.VMEM((_CH, 2 * _D), jnp.float32),
            pltpu.VMEM((_CH, 2 * _D), jnp.float32),
            pltpu.VMEM((_CH, 2 * _D), jnp.float32),
            pltpu.VMEM((_BPW,), jnp.float32),
            pltpu.SemaphoreType.DMA,
        ],
    )
    p_score, n_score = score_run(ph, pr, pt, nh, nr, nt, ent2, rel2)
    return (p_score, n_score)


# speedup vs baseline: 1.2129x; 1.0158x over previous
import jax
import jax.numpy as jnp
from jax import lax
from jax.experimental import pallas as pl
from jax.experimental.pallas import tpu as pltpu
from jax.experimental.pallas import tpu_sc as plsc

_B = 16384
_D = 64
_V = 1_000_000
_NC = 2
_NS = 16
_NW = _NC * _NS
_BPW = _B // _NW
_CH = 128
_NCH = _BPW // _CH
_NG = _CH // 16

_TCOLS = _V // 128
_TAIL = _V - _TCOLS * 128
_ITER = (_TCOLS + _NW - 1) // _NW


def _rsqrt(x):
    i = lax.bitcast_convert_type(x, jnp.int32)
    seed = jnp.int32(0x5F3759DF) - lax.shift_right_logical(i, 1)
    y = lax.bitcast_convert_type(seed, jnp.float32)
    for _ in range(3):
        y = y * (1.5 - 0.5 * x * y * y)
    return y


def _diag_bases(iot):
    zero = jnp.zeros((16,), jnp.int32)
    ib, ob = [], []
    for s in range(16):
        t = jnp.bitwise_and(iot + s, 15)
        ib.append(iot * 128 + t)
        ob.append(lax.shift_right_logical(t, 1) * 128
                  + lax.shift_left(jnp.bitwise_and(t, 1), 6) + iot)
    return zero, ib, ob


def _transpose_block(inb, outb, bases):
    zero, ib, ob = bases

    def sub(sb, carry):
        d0 = jnp.bitwise_and(sb, 3) * 16
        e0 = lax.shift_right_logical(sb, 2) * 16
        ioff = d0 * 128 + e0
        ooff = lax.shift_right_logical(e0, 1) * 128 + d0
        gs = [plsc.load_gather(inb, [zero, ib[s] + ioff]) for s in range(16)]
        for s in range(16):
            plsc.store_scatter(outb, [zero, ob[s] + ooff], gs[s])
        return carry

    lax.fori_loop(0, 32, sub, 0)


def _trans_body(ent_t, tail32, out, inbuf, outbuf,
                sin0, sin1, sin2, sin3, sin4, sin5, sin6, sin7,
                sout0, sout1, sout2, sout3):
    wid = lax.axis_index("s") * _NC + lax.axis_index("c")
    iot = lax.iota(jnp.int32, 16)
    sin = (sin0, sin1, sin2, sin3, sin4, sin5, sin6, sin7)
    sout = (sout0, sout1, sout2, sout3)
    bases = _diag_bases(iot)

    def in_copy(k, b):
        cc = wid + k * _NW

        @pl.when(cc < _TCOLS)
        def _():
            pltpu.async_copy(ent_t.at[:, pl.ds(cc * 128, 128)],
                             inbuf.at[b], sin[b])

    def in_wait(k, b):
        cc = wid + k * _NW

        @pl.when(cc < _TCOLS)
        def _():
            pltpu.make_async_copy(ent_t.at[:, pl.ds(cc * 128, 128)],
                                  inbuf.at[b], sin[b]).wait()

    def out_copy(k, b):
        cc = wid + k * _NW

        @pl.when(cc < _TCOLS)
        def _():
            pltpu.async_copy(outbuf.at[b], out.at[pl.ds(cc * 64, 64)],
                             sout[b])

    def out_wait(k, b):
        cc = wid + k * _NW

        @pl.when(cc < _TCOLS)
        def _():
            pltpu.make_async_copy(outbuf.at[b], out.at[pl.ds(cc * 64, 64)],
                                  sout[b]).wait()

    for k0 in range(6):
        in_copy(k0, k0)

    def step(i, carry):
        for b in range(8):
            k = 8 * i + b
            in_copy(k + 6, (b + 6) % 8)
            in_wait(k, b)

            @pl.when(k >= 4)
            def _():
                out_wait(k - 4, b % 4)

            @pl.when(wid + k * _NW < _TCOLS)
            def _():
                _transpose_block(inbuf.at[b], outbuf.at[b % 4], bases)

            out_copy(k, b % 4)
        return carry

    lax.fori_loop(0, (_ITER + 11) // 8, step, 0)

    @pl.when(wid == 0)
    def _():
        pltpu.sync_copy(tail32, outbuf.at[0].at[pl.ds(0, _TAIL // 2)])
        pltpu.sync_copy(outbuf.at[0].at[pl.ds(0, _TAIL // 2)],
                        out.at[pl.ds(_TCOLS * 64, _TAIL // 2)])


def _score_body(ph, pr, pt, nh, nr, nt, ent2, rel2, p_out, n_out,
                ia, ib, ic, ja, jb, jc, abuf, bbuf, cbuf, obuf, sem):
    wid = lax.axis_index("s") * _NC + lax.axis_index("c")
    iot = lax.iota(jnp.int32, 16)

    for idx_a, tab_a, idx_b, tab_b, idx_c, tab_c, out in (
        (ph, ent2, pr, rel2, pt, ent2, p_out),
        (nh, ent2, nt, ent2, nr, rel2, n_out),
    ):
        row0 = wid * _NCH

        def chunk_body(c, carry):
            pltpu.sync_copy(idx_a.at[pl.ds(row0 + c, 1)], ia)
            pltpu.sync_copy(idx_b.at[pl.ds(row0 + c, 1)], ib)
            pltpu.sync_copy(idx_c.at[pl.ds(row0 + c, 1)], ic)
            for src, dst in ((ia, ja), (ib, jb), (ic, jc)):
                for k in range(_CH // 16):
                    sl = pl.ds(k * 16, 16)
                    dst[0, sl] = lax.shift_right_logical(src[0, sl], 1)
            da = pltpu.async_copy(tab_a.at[ja.at[0]], abuf, sem)
            db = pltpu.async_copy(tab_b.at[jb.at[0]], bbuf, sem)
            dc = pltpu.async_copy(tab_c.at[jc.at[0]], cbuf, sem)
            da.wait()
            db.wait()
            dc.wait()

            def group(g, inner):
                r = g * 16 + iot
                zi = jnp.zeros((16,), jnp.int32)
                ha = jnp.bitwise_and(plsc.load_gather(ia, [zi, r]), 1) * _D
                hb = jnp.bitwise_and(plsc.load_gather(ib, [zi, r]), 1) * _D
                hc = jnp.bitwise_and(plsc.load_gather(ic, [zi, r]), 1) * _D
                r128 = r * 128
                fa = r128 + ha
                fb = r128 + hb
                fc = r128 + hc
                z = jnp.zeros((16,), jnp.float32)
                aa, bb, cc, ab, ac, bc = z, z, z, z, z, z
                for d in range(_D):
                    col = jnp.bitwise_and(iot + d, _D - 1)
                    av = plsc.load_gather(abuf, [zi, fa + col])
                    bv = plsc.load_gather(bbuf, [zi, fb + col])
                    cv = plsc.load_gather(cbuf, [zi, fc + col])
                    aa += av * av
                    bb += bv * bv
                    cc += cv * cv
                    ab += av * bv
                    ac += av * cv
                    bc += bv * cv
                inva = _rsqrt(jnp.maximum(aa, 1e-24))
                invb = _rsqrt(jnp.maximum(bb, 1e-24))
                invc = _rsqrt(jnp.maximum(cc, 1e-24))
                s2 = 3.0 + 2.0 * (ab * inva * invb - ac * inva * invc
                                  - bc * invb * invc)
                s2 = jnp.maximum(s2, 0.0)
                score = s2 * _rsqrt(jnp.maximum(s2, 1e-30))
                obuf[pl.ds(c * _CH + g * 16, 16)] = score
                return inner

            lax.fori_loop(0, _NG, group, 0)
            return carry

        lax.fori_loop(0, _NCH, chunk_body, 0)
        pltpu.sync_copy(obuf, out.at[pl.ds(wid * _BPW, _BPW)])


def kernel(pos_h, pos_r, pos_t, neg_h, neg_r, neg_t, ent_emb, rel_emb):
    shp = (_B // _CH, _CH)
    ph = pos_h.astype(jnp.int32).reshape(shp)
    pr = pos_r.astype(jnp.int32).reshape(shp)
    pt = pos_t.astype(jnp.int32).reshape(shp)
    nh = neg_h.astype(jnp.int32).reshape(shp)
    nr = neg_r.astype(jnp.int32).reshape(shp)
    nt = neg_t.astype(jnp.int32).reshape(shp)
    ent_t = jnp.transpose(ent_emb)
    tail32 = ent_emb[_TCOLS * 128:].reshape(_TAIL // 2, 2 * _D)
    rel2 = rel_emb.reshape(-1, 2 * _D)

    mesh = plsc.VectorSubcoreMesh(core_axis_name="c", subcore_axis_name="s")
    cparams = pltpu.CompilerParams(
        use_tc_tiling_on_sc=True,
        needs_layout_passes=False,
        disable_bounds_checks=True,
    )

    transpose_run = pl.kernel(
        _trans_body,
        mesh=mesh,
        compiler_params=cparams,
        out_type=[jax.ShapeDtypeStruct((_V // 2, 2 * _D), jnp.float32)],
        scratch_types=(
            [pltpu.VMEM((8, _D, 128), jnp.float32),
             pltpu.VMEM((4, _D, 128), jnp.float32)]
            + [pltpu.SemaphoreType.DMA] * 12
        ),
    )
    (ent2,) = transpose_run(ent_t, tail32)

    score_run = pl.kernel(
        _score_body,
        mesh=mesh,
        compiler_params=cparams,
        out_type=[
            jax.ShapeDtypeStruct((_B,), jnp.float32),
            jax.ShapeDtypeStruct((_B,), jnp.float32),
        ],
        scratch_types=[
            pltpu.VMEM((1, _CH), jnp.int32),
            pltpu.VMEM((1, _CH), jnp.int32),
            pltpu.VMEM((1, _CH), jnp.int32),
            pltpu.VMEM((1, _CH), jnp.int32),
            pltpu.VMEM((1, _CH), jnp.int32),
            pltpu.VMEM((1, _CH), jnp.int32),
            pltpu.VMEM((_CH, 2 * _D), jnp.float32),
            pltpu.VMEM((_CH, 2 * _D), jnp.float32),
            pltpu.VMEM((_CH, 2 * _D), jnp.float32),
            pltpu.VMEM((_BPW,), jnp.float32),
            pltpu.SemaphoreType.DMA,
        ],
    )
    p_score, n_score = score_run(ph, pr, pt, nh, nr, nt, ent2, rel2)
    return (p_score, n_score)


# speedup vs baseline: 1.2921x; 1.0653x over previous
import jax
import jax.numpy as jnp
from jax import lax
from jax.experimental import pallas as pl
from jax.experimental.pallas import tpu as pltpu
from jax.experimental.pallas import tpu_sc as plsc

_B = 16384
_D = 64
_V = 1_000_000
_NC = 2
_NS = 16
_NW = _NC * _NS
_BPW = _B // _NW
_CH = 128
_NCH = _BPW // _CH
_NG = _CH // 16

_TCOLS = _V // 128
_TAIL = _V - _TCOLS * 128
_ITER = (_TCOLS + _NW - 1) // _NW


def _rsqrt(x):
    i = lax.bitcast_convert_type(x, jnp.int32)
    seed = jnp.int32(0x5F3759DF) - lax.shift_right_logical(i, 1)
    y = lax.bitcast_convert_type(seed, jnp.float32)
    for _ in range(3):
        y = y * (1.5 - 0.5 * x * y * y)
    return y


def _diag_bases(iot):
    zero = jnp.zeros((16,), jnp.int32)
    ib, ob = [], []
    for s in range(16):
        t = jnp.bitwise_and(iot + s, 15)
        ib.append(iot * 128 + t)
        ob.append(lax.shift_right_logical(t, 1) * 128
                  + lax.shift_left(jnp.bitwise_and(t, 1), 6) + iot)
    return zero, ib, ob


def _transpose_block(inb, outb, bases):
    zero, ib, ob = bases

    def sub(sb, carry):
        d0 = jnp.bitwise_and(sb, 3) * 16
        e0 = lax.shift_right_logical(sb, 2) * 16
        ioff = d0 * 128 + e0
        ooff = lax.shift_right_logical(e0, 1) * 128 + d0
        gs = [plsc.load_gather(inb, [zero, ib[s] + ioff]) for s in range(16)]
        for s in range(16):
            plsc.store_scatter(outb, [zero, ob[s] + ooff], gs[s])
        return carry

    lax.fori_loop(0, 32, sub, 0)


def _trans_body(ent_t, tail32, out, inbuf, outbuf,
                sin0, sin1, sin2, sin3, sin4, sin5, sin6, sin7,
                sout0, sout1, sout2, sout3):
    wid = lax.axis_index("s") * _NC + lax.axis_index("c")
    iot = lax.iota(jnp.int32, 16)
    sin = (sin0, sin1, sin2, sin3, sin4, sin5, sin6, sin7)
    sout = (sout0, sout1, sout2, sout3)
    bases = _diag_bases(iot)

    def in_copy(k, b):
        cc = wid + k * _NW

        @pl.when(cc < _TCOLS)
        def _():
            pltpu.async_copy(ent_t.at[:, pl.ds(cc * 128, 128)],
                             inbuf.at[b], sin[b])

    def in_wait(k, b):
        cc = wid + k * _NW

        @pl.when(cc < _TCOLS)
        def _():
            pltpu.make_async_copy(ent_t.at[:, pl.ds(cc * 128, 128)],
                                  inbuf.at[b], sin[b]).wait()

    def out_copy(k, b):
        cc = wid + k * _NW

        @pl.when(cc < _TCOLS)
        def _():
            pltpu.async_copy(outbuf.at[b], out.at[pl.ds(cc * 64, 64)],
                             sout[b])

    def out_wait(k, b):
        cc = wid + k * _NW

        @pl.when(cc < _TCOLS)
        def _():
            pltpu.make_async_copy(outbuf.at[b], out.at[pl.ds(cc * 64, 64)],
                                  sout[b]).wait()

    for k0 in range(6):
        in_copy(k0, k0)

    def step(i, carry):
        for b in range(8):
            k = 8 * i + b
            in_copy(k + 6, (b + 6) % 8)
            in_wait(k, b)

            @pl.when(k >= 4)
            def _():
                out_wait(k - 4, b % 4)

            @pl.when(wid + k * _NW < _TCOLS)
            def _():
                _transpose_block(inbuf.at[b], outbuf.at[b % 4], bases)

            out_copy(k, b % 4)
        return carry

    lax.fori_loop(0, (_ITER + 11) // 8, step, 0)

    @pl.when(wid == 0)
    def _():
        pltpu.sync_copy(tail32, outbuf.at[0].at[pl.ds(0, _TAIL // 2)])
        pltpu.sync_copy(outbuf.at[0].at[pl.ds(0, _TAIL // 2)],
                        out.at[pl.ds(_TCOLS * 64, _TAIL // 2)])


def _score_body(pha, phh, pra, prh, pta, pth, nha, nhh, nta, nth, nra, nrh,
                ent2, rel2, p_out, n_out,
                jpa, jpb, jpc, iah, ibh, ich,
                abuf, bbuf, cbuf, obuf, sidx, srow0, srow1):
    wid = lax.axis_index("s") * _NC + lax.axis_index("c")
    iot = lax.iota(jnp.int32, 16)
    srow = (srow0, srow1)
    row0 = wid * _NCH

    phases = (
        (pha, phh, ent2, pra, prh, rel2, pta, pth, ent2, p_out),
        (nha, nhh, ent2, nta, nth, ent2, nra, nrh, rel2, n_out),
    )

    idx_dmas = []
    for p, (ja, ha_in, _ta, jb, hb_in, _tb, jc, hc_in, _tc, _o) in enumerate(phases):
        for c in range(_NCH):
            s = p * _NCH + c
            for src, dst in ((ja, jpa), (jb, jpb), (jc, jpc),
                             (ha_in, iah), (hb_in, ibh), (hc_in, ich)):
                idx_dmas.append(pltpu.async_copy(
                    src.at[pl.ds(row0 + c, 1)], dst.at[pl.ds(s, 1)], sidx))
    for dma in idx_dmas:
        dma.wait()

    def fire_rows(p, c):
        s = p * _NCH + c
        tabs = (phases[p][2], phases[p][5], phases[p][8])
        for tab, jp, buf in ((tabs[0], jpa, abuf), (tabs[1], jpb, bbuf),
                             (tabs[2], jpc, cbuf)):
            pltpu.async_copy(tab.at[jp.at[s]], buf.at[s % 2], srow[s % 2])

    def wait_rows(p, c):
        s = p * _NCH + c
        tabs = (phases[p][2], phases[p][5], phases[p][8])
        for tab, jp, buf in ((tabs[0], jpa, abuf), (tabs[1], jpb, bbuf),
                             (tabs[2], jpc, cbuf)):
            pltpu.make_async_copy(tab.at[jp.at[s]], buf.at[s % 2],
                                  srow[s % 2]).wait()

    fire_rows(0, 0)
    zi = jnp.zeros((16,), jnp.int32)
    for p in range(2):
        for c in range(_NCH):
            s = p * _NCH + c
            if s + 1 < 2 * _NCH:
                fire_rows((s + 1) // _NCH, (s + 1) % _NCH)
            wait_rows(p, c)
            par = s % 2
            ab_ = abuf.at[par]
            bb_ = bbuf.at[par]
            cb_ = cbuf.at[par]

            def group(g, inner):
                r = g * 16 + iot
                soff = s * 128
                ha = plsc.load_gather(iah, [zi, soff + r])
                hb = plsc.load_gather(ibh, [zi, soff + r])
                hc = plsc.load_gather(ich, [zi, soff + r])
                r128 = r * 128
                fa = r128 + ha
                fb = r128 + hb
                fc = r128 + hc
                z = jnp.zeros((16,), jnp.float32)
                aa, bb, cc, ab, ac, bc = z, z, z, z, z, z
                for d in range(_D):
                    col = jnp.bitwise_and(iot + d, _D - 1)
                    av = plsc.load_gather(ab_, [zi, fa + col])
                    bv = plsc.load_gather(bb_, [zi, fb + col])
                    cv = plsc.load_gather(cb_, [zi, fc + col])
                    aa += av * av
                    bb += bv * bv
                    cc += cv * cv
                    ab += av * bv
                    ac += av * cv
                    bc += bv * cv
                inva = _rsqrt(jnp.maximum(aa, 1e-24))
                invb = _rsqrt(jnp.maximum(bb, 1e-24))
                invc = _rsqrt(jnp.maximum(cc, 1e-24))
                s2 = 3.0 + 2.0 * (ab * inva * invb - ac * inva * invc
                                  - bc * invb * invc)
                s2 = jnp.maximum(s2, 0.0)
                score = s2 * _rsqrt(jnp.maximum(s2, 1e-30))
                obuf[pl.ds(c * _CH + g * 16, 16)] = score
                return inner

            lax.fori_loop(0, _NG, group, 0)
        pltpu.sync_copy(obuf, phases[p][9].at[pl.ds(wid * _BPW, _BPW)])


def kernel(pos_h, pos_r, pos_t, neg_h, neg_r, neg_t, ent_emb, rel_emb):
    shp = (_B // _CH, _CH)

    def prep(x):
        x = x.astype(jnp.int32)
        pair = lax.shift_right_logical(x, 1).reshape(shp)
        half = (jnp.bitwise_and(x, 1) * _D).reshape(shp)
        return pair, half

    pha, phh = prep(pos_h)
    pra, prh = prep(pos_r)
    pta, pth = prep(pos_t)
    nha, nhh = prep(neg_h)
    nra, nrh = prep(neg_r)
    nta, nth = prep(neg_t)
    ent_t = jnp.transpose(ent_emb)
    tail32 = ent_emb[_TCOLS * 128:].reshape(_TAIL // 2, 2 * _D)
    rel2 = rel_emb.reshape(-1, 2 * _D)

    mesh = plsc.VectorSubcoreMesh(core_axis_name="c", subcore_axis_name="s")
    cparams = pltpu.CompilerParams(
        use_tc_tiling_on_sc=True,
        needs_layout_passes=False,
        disable_bounds_checks=True,
    )

    transpose_run = pl.kernel(
        _trans_body,
        mesh=mesh,
        compiler_params=cparams,
        out_type=[jax.ShapeDtypeStruct((_V // 2, 2 * _D), jnp.float32)],
        scratch_types=(
            [pltpu.VMEM((8, _D, 128), jnp.float32),
             pltpu.VMEM((4, _D, 128), jnp.float32)]
            + [pltpu.SemaphoreType.DMA] * 12
        ),
    )
    (ent2,) = transpose_run(ent_t, tail32)

    score_run = pl.kernel(
        _score_body,
        mesh=mesh,
        compiler_params=cparams,
        out_type=[
            jax.ShapeDtypeStruct((_B,), jnp.float32),
            jax.ShapeDtypeStruct((_B,), jnp.float32),
        ],
        scratch_types=(
            [pltpu.VMEM((2 * _NCH, _CH), jnp.int32)] * 6
            + [pltpu.VMEM((2, _CH, 2 * _D), jnp.float32)] * 3
            + [pltpu.VMEM((_BPW,), jnp.float32)]
            + [pltpu.SemaphoreType.DMA] * 3
        ),
    )
    p_score, n_score = score_run(pha, phh, pra, prh, pta, pth,
                                 nha, nhh, nta, nth, nra, nrh, ent2, rel2)
    return (p_score, n_score)
